# baseline (speedup 1.0000x reference)
import functools

import jax
import jax.numpy as jnp
from jax import lax
from jax.experimental import pallas as pl
from jax.experimental.pallas import tpu as pltpu
from jax.experimental.pallas import tpu_sc as plsc

NODES = 10000
EDGES = 320000
FEAT = 128

NC, NS = 2, 16
CH = 24
EPT = EDGES // NS
NCH = -(-EPT // CH)
NCH = -(-NCH // 8) * 8
PAD = NCH * CH - EPT
XROWS = 10016
XBLK = 640
SPLIT = 5056
ACCR = 5248
RPT = ACCR // NS
NSPARE0 = ACCR - SPLIT
NSPARE1 = ACCR - (NODES - SPLIT)
IDEP = 8
PDIST = 6


def _sc_body(x_hbm, idx_hbm, zero_hbm, out0, out1,
             xs, acc, ibuf, r0, r1, g0, g1, t0, t1, *isems):
    rows = (r0, r1)
    gsems = (g0, g1)
    ssems = (t0, t1)
    c = lax.axis_index("c")
    s = lax.axis_index("s")

    pltpu.sync_copy(zero_hbm.at[pl.ds(s * RPT, RPT)],
                    acc.at[pl.ds(s * RPT, RPT)])

    @pl.when(s < NS - 1)
    def _():
        pltpu.sync_copy(x_hbm.at[pl.ds(s * XBLK, XBLK)],
                        xs.at[pl.ds(s * XBLK, XBLK)])

    @pl.when(s == NS - 1)
    def _():
        blk = XROWS - (NS - 1) * XBLK
        pltpu.sync_copy(x_hbm.at[pl.ds((NS - 1) * XBLK, blk)],
                        xs.at[pl.ds((NS - 1) * XBLK, blk)])

    plsc.subcore_barrier()

    cbase = (c * NS + s) * NCH

    for i in range(PDIST):
        pltpu.async_copy(idx_hbm.at[cbase + i], ibuf.at[pl.ds(2 * i, 2)],
                         isems[i])

    def body(J, carry):
        j0 = J * IDEP
        for half in range(IDEP // 2):
            pair = (2 * half, 2 * half + 1)
            for g in pair:
                b = g % 2
                j = j0 + g
                @pl.when(j >= 2)
                def _(b=b, g=g):
                    pltpu.make_async_copy(
                        rows[b], acc.at[ibuf.at[2 * ((g - 2) % IDEP) + 1]],
                        ssems[b]).wait()

                @pl.when(j + PDIST < NCH)
                def _(g=g, j=j):
                    pltpu.async_copy(
                        idx_hbm.at[cbase + j + PDIST],
                        ibuf.at[pl.ds(2 * ((g + PDIST) % IDEP), 2)],
                        isems[(g + PDIST) % IDEP])

                pltpu.make_async_copy(idx_hbm.at[cbase + j],
                                      ibuf.at[pl.ds(2 * g, 2)],
                                      isems[g]).wait()
                pltpu.async_copy(xs.at[ibuf.at[2 * g]], rows[b], gsems[b])
            for g in pair:
                b = g % 2
                pltpu.make_async_copy(xs.at[ibuf.at[2 * g]], rows[b],
                                      gsems[b]).wait()
                pltpu.async_copy(rows[b], acc.at[ibuf.at[2 * g + 1]],
                                 ssems[b], add=True)
        return carry

    lax.fori_loop(0, NCH // IDEP, body, 0)
    for g in (IDEP - 2, IDEP - 1):
        pltpu.make_async_copy(rows[g % 2], acc.at[ibuf.at[2 * g + 1]],
                              ssems[g % 2]).wait()
    plsc.subcore_barrier()

    @pl.when(c == 0)
    def _():
        pltpu.sync_copy(acc.at[pl.ds(s * RPT, RPT)],
                        out0.at[pl.ds(s * RPT, RPT)])

    @pl.when(c == 1)
    def _():
        pltpu.sync_copy(acc.at[pl.ds(s * RPT, RPT)],
                        out1.at[pl.ds(s * RPT, RPT)])


_sc_scatter = functools.partial(
    pl.kernel,
    mesh=plsc.VectorSubcoreMesh(core_axis_name="c", subcore_axis_name="s"),
    out_type=(
        jax.ShapeDtypeStruct((ACCR, FEAT), jnp.float32),
        jax.ShapeDtypeStruct((ACCR, FEAT), jnp.float32),
    ),
    scratch_types=[
        pltpu.VMEM_SHARED((XROWS, FEAT), jnp.float32),
        pltpu.VMEM_SHARED((ACCR, FEAT), jnp.float32),
        pltpu.VMEM((2 * IDEP, CH), jnp.int32),
        pltpu.VMEM((CH, FEAT), jnp.float32),
        pltpu.VMEM((CH, FEAT), jnp.float32),
        pltpu.SemaphoreType.DMA,
        pltpu.SemaphoreType.DMA,
        pltpu.SemaphoreType.DMA,
        pltpu.SemaphoreType.DMA,
    ] + [pltpu.SemaphoreType.DMA for _ in range(IDEP)],
)(_sc_body)


_BLK = 16
_NB0 = SPLIT // _BLK


def _mix_body(a_ref, b_ref, o_ref):
    o_ref[...] = jnp.where(pl.program_id(0) < _NB0, a_ref[...], b_ref[...])


_tc_mix = pl.pallas_call(
    _mix_body,
    grid=(NODES // _BLK,),
    in_specs=[
        pl.BlockSpec((_BLK, FEAT), lambda i: (jnp.minimum(i, _NB0 - 1), 0)),
        pl.BlockSpec((_BLK, FEAT),
                     lambda i: (jnp.maximum(i - _NB0, 0), 0)),
    ],
    out_specs=pl.BlockSpec((_BLK, FEAT), lambda i: (i, 0)),
    out_shape=jax.ShapeDtypeStruct((NODES, FEAT), jnp.float32),
)


def kernel(x, edge_index):
    src = edge_index[0]
    tgt = edge_index[1]
    eid = jnp.arange(EDGES, dtype=jnp.int32)
    tgt0 = jnp.where(tgt < SPLIT, tgt, SPLIT + eid % NSPARE0)
    tgt1 = jnp.where(tgt >= SPLIT, tgt - SPLIT,
                     (NODES - SPLIT) + eid % NSPARE1)
    pid = jnp.arange(PAD, dtype=jnp.int32)
    fsrc = jnp.broadcast_to(pid % NODES, (NS, PAD))
    f0 = jnp.broadcast_to(SPLIT + pid % NSPARE0, (NS, PAD))
    f1 = jnp.broadcast_to((NODES - SPLIT) + pid % NSPARE1, (NS, PAD))
    srcp = jnp.concatenate([src.reshape(NS, EPT), fsrc], axis=1)
    tgt0p = jnp.concatenate([tgt0.reshape(NS, EPT), f0], axis=1)
    tgt1p = jnp.concatenate([tgt1.reshape(NS, EPT), f1], axis=1)
    srcc = srcp.reshape(NS, NCH, 1, CH)
    idx = jnp.stack([
        jnp.concatenate([srcc, tgt0p.reshape(NS, NCH, 1, CH)], axis=2),
        jnp.concatenate([srcc, tgt1p.reshape(NS, NCH, 1, CH)], axis=2),
    ]).reshape(NC * NS * NCH, 2, CH)
    xp = jnp.pad(x, ((0, XROWS - NODES), (0, 0)))
    zero = jnp.zeros((ACCR, FEAT), jnp.float32)
    p0, p1 = _sc_scatter(xp, idx, zero)
    return _tc_mix(p0[:SPLIT], p1[:NODES - SPLIT])


# speedup vs baseline: 1.3298x; 1.3298x over previous
import functools

import jax
import jax.numpy as jnp
from jax import lax
from jax.experimental import pallas as pl
from jax.experimental.pallas import tpu as pltpu
from jax.experimental.pallas import tpu_sc as plsc

NODES = 10000
EDGES = 320000
FEAT = 128

NC, NS = 2, 16
NW = NC * NS
CH = 128
EPW = EDGES // NW
CPW = -(-EPW // CH)
CPW = CPW + (CPW % 2)
PAD = CPW * CH - EPW
RPT = 632
NODES_PAD = NS * RPT
DEPTH = 2
SEG = 2
CPS = CPW // SEG


def _sc_body(x_hbm, src_hbm, tgt_hbm, zero_hbm, out0, out1,
             acc, sidx, tidx, r0, r1, g0, g1, t0, t1):
    gsems = (g0, g1)
    ssems = (t0, t1)
    c = lax.axis_index("c")
    s = lax.axis_index("s")
    wid = s * NC + c

    row0 = s * RPT
    pltpu.sync_copy(zero_hbm.at[pl.ds(row0, RPT)], acc.at[pl.ds(row0, RPT)])

    plsc.subcore_barrier()

    cbase = wid * CPW
    rows = (r0, r1)

    for seg in range(SEG):
        pltpu.sync_copy(src_hbm.at[pl.ds(cbase + seg * CPS, CPS)], sidx)
        pltpu.sync_copy(tgt_hbm.at[pl.ds(cbase + seg * CPS, CPS)], tidx)

        def body(j, carry):
            k = j * DEPTH
            for b in range(DEPTH):
                @pl.when(j > 0)
                def _(b=b, k=k):
                    pltpu.make_async_copy(
                        rows[b], acc.at[tidx.at[k + b - DEPTH]],
                        ssems[b]).wait()
                pltpu.async_copy(x_hbm.at[sidx.at[k + b]], rows[b], gsems[b])
            for b in range(DEPTH):
                pltpu.make_async_copy(x_hbm.at[sidx.at[k + b]], rows[b],
                                      gsems[b]).wait()
                pltpu.async_copy(rows[b], acc.at[tidx.at[k + b]], ssems[b],
                                 add=True)
            return carry

        lax.fori_loop(0, CPS // DEPTH, body, 0)
        for b in range(DEPTH):
            pltpu.make_async_copy(rows[b], acc.at[tidx.at[CPS - DEPTH + b]],
                                  ssems[b]).wait()
    plsc.subcore_barrier()

    @pl.when(c == 0)
    def _():
        pltpu.sync_copy(acc.at[pl.ds(row0, RPT)], out0.at[pl.ds(row0, RPT)])

    @pl.when(c == 1)
    def _():
        pltpu.sync_copy(acc.at[pl.ds(row0, RPT)], out1.at[pl.ds(row0, RPT)])


_sc_scatter = functools.partial(
    pl.kernel,
    mesh=plsc.VectorSubcoreMesh(core_axis_name="c", subcore_axis_name="s"),
    out_type=(
        jax.ShapeDtypeStruct((NODES_PAD, FEAT), jnp.float32),
        jax.ShapeDtypeStruct((NODES_PAD, FEAT), jnp.float32),
    ),
    scratch_types=[
        pltpu.VMEM_SHARED((NODES_PAD, FEAT), jnp.float32),
        pltpu.VMEM((CPS, CH), jnp.int32),
        pltpu.VMEM((CPS, CH), jnp.int32),
        pltpu.VMEM((CH, FEAT), jnp.float32),
        pltpu.VMEM((CH, FEAT), jnp.float32),
        pltpu.SemaphoreType.DMA,
        pltpu.SemaphoreType.DMA,
        pltpu.SemaphoreType.DMA,
        pltpu.SemaphoreType.DMA,
    ],
)(_sc_body)


def _add_body(a_ref, b_ref, o_ref):
    o_ref[...] = a_ref[...] + b_ref[...]


_tc_add = pl.pallas_call(
    _add_body,
    grid=(NS,),
    in_specs=[
        pl.BlockSpec((RPT, FEAT), lambda i: (i, 0)),
        pl.BlockSpec((RPT, FEAT), lambda i: (i, 0)),
    ],
    out_specs=pl.BlockSpec((RPT, FEAT), lambda i: (i, 0)),
    out_shape=jax.ShapeDtypeStruct((NODES_PAD, FEAT), jnp.float32),
)


def kernel(x, edge_index):
    src = edge_index[0].reshape(NW, EPW)
    tgt = edge_index[1].reshape(NW, EPW)
    src = jnp.pad(src, ((0, 0), (0, PAD))).reshape(NW * CPW, CH)
    tpad = NODES + jnp.arange(PAD, dtype=jnp.int32) % 16
    tgt = jnp.concatenate(
        [tgt, jnp.broadcast_to(tpad, (NW, PAD))], axis=1
    ).reshape(NW * CPW, CH)
    zero = jnp.zeros((NODES_PAD, FEAT), jnp.float32)
    p0, p1 = _sc_scatter(x, src, tgt, zero)
    return _tc_add(p0, p1)[:NODES]


# speedup vs baseline: 3.2147x; 2.4175x over previous
import functools

import jax
import jax.numpy as jnp
from jax import lax
from jax.experimental import pallas as pl
from jax.experimental.pallas import tpu as pltpu
from jax.experimental.pallas import tpu_sc as plsc

NODES = 10000
EDGES = 320000
FEAT = 128

NC, NS = 2, 16
NW = NC * NS
CH = 128
EPW = EDGES // NW
CPW = -(-EPW // CH)
CPW = CPW + (CPW % 2)
PAD = CPW * CH - EPW
RPT = 632
NODES_PAD = NS * RPT
DEPTH = 2
SEG = 2
CPS = CPW // SEG


def _sc_body(x_hbm, src_hbm, tgt_hbm, zero_hbm, out0, out1,
             acc, sidx, tidx, r0, r1, g0, g1, t0, t1):
    gsems = (g0, g1)
    ssems = (t0, t1)
    c = lax.axis_index("c")
    s = lax.axis_index("s")
    wid = s * NC + c

    row0 = s * RPT
    pltpu.sync_copy(zero_hbm.at[pl.ds(row0, RPT)], acc.at[pl.ds(row0, RPT)])

    plsc.subcore_barrier()

    cbase = wid * CPW
    rows = (r0, r1)

    for seg in range(SEG):
        pltpu.sync_copy(src_hbm.at[pl.ds(cbase + seg * CPS, CPS)], sidx)
        pltpu.sync_copy(tgt_hbm.at[pl.ds(cbase + seg * CPS, CPS)], tidx)

        def body(j, carry):
            k = j * DEPTH
            for b in range(DEPTH):
                @pl.when(j > 0)
                def _(b=b, k=k):
                    pltpu.make_async_copy(
                        rows[b], acc.at[tidx.at[k + b - DEPTH]],
                        ssems[b]).wait()
                pltpu.async_copy(x_hbm.at[sidx.at[k + b]], rows[b], gsems[b])
            for b in range(DEPTH):
                pltpu.make_async_copy(x_hbm.at[sidx.at[k + b]], rows[b],
                                      gsems[b]).wait()
                pltpu.async_copy(rows[b], acc.at[tidx.at[k + b]], ssems[b],
                                 add=True)
            return carry

        lax.fori_loop(0, CPS // DEPTH, body, 0)
        for b in range(DEPTH):
            pltpu.make_async_copy(rows[b], acc.at[tidx.at[CPS - DEPTH + b]],
                                  ssems[b]).wait()
    plsc.subcore_barrier()

    @pl.when(c == 0)
    def _():
        pltpu.sync_copy(acc.at[pl.ds(row0, RPT)], out0.at[pl.ds(row0, RPT)])

    @pl.when(c == 1)
    def _():
        pltpu.sync_copy(acc.at[pl.ds(row0, RPT)], out1.at[pl.ds(row0, RPT)])


_sc_scatter = functools.partial(
    pl.kernel,
    mesh=plsc.VectorSubcoreMesh(core_axis_name="c", subcore_axis_name="s"),
    out_type=(
        jax.ShapeDtypeStruct((NODES_PAD, FEAT), jnp.float32),
        jax.ShapeDtypeStruct((NODES_PAD, FEAT), jnp.float32),
    ),
    scratch_types=[
        pltpu.VMEM_SHARED((NODES_PAD, FEAT), jnp.float32),
        pltpu.VMEM((CPS, CH), jnp.int32),
        pltpu.VMEM((CPS, CH), jnp.int32),
        pltpu.VMEM((CH, FEAT), jnp.float32),
        pltpu.VMEM((CH, FEAT), jnp.float32),
        pltpu.SemaphoreType.DMA,
        pltpu.SemaphoreType.DMA,
        pltpu.SemaphoreType.DMA,
        pltpu.SemaphoreType.DMA,
    ],
)(_sc_body)


def _add_body(a_ref, b_ref, o_ref):
    o_ref[...] = a_ref[...] + b_ref[...]


_tc_add = pl.pallas_call(
    _add_body,
    grid=(NS,),
    in_specs=[
        pl.BlockSpec((RPT, FEAT), lambda i: (i, 0)),
        pl.BlockSpec((RPT, FEAT), lambda i: (i, 0)),
    ],
    out_specs=pl.BlockSpec((RPT, FEAT), lambda i: (i, 0)),
    out_shape=jax.ShapeDtypeStruct((NODES_PAD, FEAT), jnp.float32),
)


def kernel(x, edge_index):
    src = edge_index[0].reshape(NW, EPW)
    tgt = edge_index[1].reshape(NW, EPW)
    pid = jnp.arange(PAD, dtype=jnp.int32)[None, :]
    wid = jnp.arange(NW, dtype=jnp.int32)[:, None]
    spad = (wid * PAD + pid) % NODES
    src = jnp.concatenate([src, spad], axis=1).reshape(NW * CPW, CH)
    tpad = NODES + (wid * PAD + pid) % (NODES_PAD - NODES)
    tgt = jnp.concatenate([tgt, tpad], axis=1).reshape(NW * CPW, CH)
    zero = jnp.zeros((NODES_PAD, FEAT), jnp.float32)
    p0, p1 = _sc_scatter(x, src, tgt, zero)
    return _tc_add(p0, p1)[:NODES]


# speedup vs baseline: 3.6360x; 1.1311x over previous
import functools

import jax
import jax.numpy as jnp
from jax import lax
from jax.experimental import pallas as pl
from jax.experimental.pallas import tpu as pltpu
from jax.experimental.pallas import tpu_sc as plsc

NODES = 10000
EDGES = 320000
FEAT = 128

NC, NS = 2, 16
NW = NC * NS
CH = 64
EPW = EDGES // NW
DEPTH = 4
CPW = -(-EPW // CH)
CPW = -(-CPW // DEPTH) * DEPTH
PAD = CPW * CH - EPW
RPT = 632
NODES_PAD = NS * RPT
SEG = 4
CPS = CPW // SEG


def _sc_body(x_hbm, src_hbm, tgt_hbm, zero_hbm, out0, out1,
             acc, sidx, tidx, *bufs):
    rows = bufs[:DEPTH]
    gsems = bufs[DEPTH:2 * DEPTH]
    ssems = bufs[2 * DEPTH:3 * DEPTH]
    c = lax.axis_index("c")
    s = lax.axis_index("s")
    wid = s * NC + c

    row0 = s * RPT
    pltpu.sync_copy(zero_hbm.at[pl.ds(row0, RPT)], acc.at[pl.ds(row0, RPT)])

    plsc.subcore_barrier()

    cbase = wid * CPW

    for seg in range(SEG):
        pltpu.sync_copy(src_hbm.at[pl.ds(cbase + seg * CPS, CPS)], sidx)
        pltpu.sync_copy(tgt_hbm.at[pl.ds(cbase + seg * CPS, CPS)], tidx)

        def body(j, carry):
            k = j * DEPTH
            for b in range(DEPTH):
                @pl.when(j > 0)
                def _(b=b, k=k):
                    pltpu.make_async_copy(
                        rows[b], acc.at[tidx.at[k + b - DEPTH]],
                        ssems[b]).wait()
                pltpu.async_copy(x_hbm.at[sidx.at[k + b]], rows[b], gsems[b])
            for b in range(DEPTH):
                pltpu.make_async_copy(x_hbm.at[sidx.at[k + b]], rows[b],
                                      gsems[b]).wait()
                pltpu.async_copy(
                    rows[b], acc.at[tidx.at[k + b]], ssems[b], add=True)
            return carry

        lax.fori_loop(0, CPS // DEPTH, body, 0)
        for b in range(DEPTH):
            pltpu.make_async_copy(
                rows[b], acc.at[tidx.at[CPS - DEPTH + b]], ssems[b]).wait()
    plsc.subcore_barrier()

    @pl.when(c == 0)
    def _():
        pltpu.sync_copy(acc.at[pl.ds(row0, RPT)], out0.at[pl.ds(row0, RPT)])

    @pl.when(c == 1)
    def _():
        pltpu.sync_copy(acc.at[pl.ds(row0, RPT)], out1.at[pl.ds(row0, RPT)])


_sc_scatter = functools.partial(
    pl.kernel,
    mesh=plsc.VectorSubcoreMesh(core_axis_name="c", subcore_axis_name="s"),
    out_type=(
        jax.ShapeDtypeStruct((NODES_PAD, FEAT), jnp.float32),
        jax.ShapeDtypeStruct((NODES_PAD, FEAT), jnp.float32),
    ),
    scratch_types=[
        pltpu.VMEM_SHARED((NODES_PAD, FEAT), jnp.float32),
        pltpu.VMEM((CPS, CH), jnp.int32),
        pltpu.VMEM((CPS, CH), jnp.int32),
    ] + [pltpu.VMEM((CH, FEAT), jnp.float32) for _ in range(DEPTH)]
      + [pltpu.SemaphoreType.DMA for _ in range(2 * DEPTH)],
)(_sc_body)


def _add_body(a_ref, b_ref, o_ref):
    o_ref[...] = a_ref[...] + b_ref[...]


_tc_add = pl.pallas_call(
    _add_body,
    grid=(NS,),
    in_specs=[
        pl.BlockSpec((RPT, FEAT), lambda i: (i, 0)),
        pl.BlockSpec((RPT, FEAT), lambda i: (i, 0)),
    ],
    out_specs=pl.BlockSpec((RPT, FEAT), lambda i: (i, 0)),
    out_shape=jax.ShapeDtypeStruct((NODES_PAD, FEAT), jnp.float32),
)


def kernel(x, edge_index):
    src = edge_index[0].reshape(NW, EPW)
    tgt = edge_index[1].reshape(NW, EPW)
    pid = jnp.arange(PAD, dtype=jnp.int32)[None, :]
    wid = jnp.arange(NW, dtype=jnp.int32)[:, None]
    spad = (wid * PAD + pid) % NODES
    src = jnp.concatenate([src, spad], axis=1).reshape(NW * CPW, CH)
    tpad = NODES + (wid * PAD + pid) % (NODES_PAD - NODES)
    tgt = jnp.concatenate([tgt, tpad], axis=1).reshape(NW * CPW, CH)
    zero = jnp.zeros((NODES_PAD, FEAT), jnp.float32)
    p0, p1 = _sc_scatter(x, src, tgt, zero)
    return _tc_add(p0, p1)[:NODES]
